# baseline (speedup 1.0000x reference)
import jax
import jax.numpy as jnp
from jax.experimental import pallas as pl

_LB = 8
_BB = 16384


def _body(xt_ref, emb_ref, o_ref):
    xv = xt_ref[...]
    emb = emb_ref[...]
    xf = xv.astype(jnp.float32)
    e0, e1, e2, e3 = emb[0], emb[1], emb[2], emb[3]
    p0 = e0
    p1 = (-11.0 * e0 + 18.0 * e1 - 9.0 * e2 + 2.0 * e3) / 6.0
    p2 = (2.0 * e0 - 5.0 * e1 + 4.0 * e2 - e3) / 2.0
    p3 = (-e0 + 3.0 * e1 - 3.0 * e2 + e3) / 6.0
    for cc in range(4):
        v = ((p3[cc] * xf + p2[cc]) * xf + p1[cc]) * xf + p0[cc]
        for k in range(8):
            o_ref[:, 4 * k + cc, :] = v


def kernel(x, emb_weight):
    B, L = x.shape
    xT = x.T
    grid = (L // _LB, B // _BB)
    out = pl.pallas_call(
        _body,
        grid=grid,
        in_specs=[
            pl.BlockSpec((_LB, _BB), lambda i, j: (i, j)),
            pl.BlockSpec((4, 4), lambda i, j: (0, 0)),
        ],
        out_specs=pl.BlockSpec((_LB, 32, _BB), lambda i, j: (i, 0, j)),
        out_shape=jax.ShapeDtypeStruct((L, 32, B), jnp.float32),
    )(xT, emb_weight)
    return jnp.transpose(out, (2, 0, 1))
